# baseline (speedup 1.0000x reference)
import functools

import jax
import jax.numpy as jnp
from jax import lax
from jax.experimental import pallas as pl
from jax.experimental.pallas import tpu as pltpu
from jax.experimental.pallas import tpu_sc as plsc

NUM_EMB = 100000
DIM = 300
B = 4096
T = 200

_info = plsc.get_sparse_core_info()
_NC, _NS = _info.num_cores, _info.num_subcores
_NW = _NC * _NS

_QROWS = 782
_M = 200
_SUB = 32
_NQ = 4
_MQ = _M // _NQ
_PLANES_MAX = -(-DIM // _NW)


def _sc_plane_gather(xi, t3):
    mesh = plsc.VectorSubcoreMesh(core_axis_name="c", subcore_axis_name="s")

    @functools.partial(
        pl.kernel,
        out_type=jax.ShapeDtypeStruct((DIM, _M, _SUB, 128), jnp.float32),
        mesh=mesh,
        scratch_types=[
            pltpu.VMEM_SHARED((_MQ, _SUB, 128), jnp.int32),
            pltpu.VMEM((_QROWS, 128), jnp.float32),
            pltpu.VMEM((_SUB, 128), jnp.int32),
            pltpu.VMEM((_SUB, 128), jnp.int32),
            pltpu.VMEM((_SUB, 128), jnp.float32),
            pltpu.VMEM((_SUB, 128), jnp.float32),
            pltpu.SemaphoreType.DMA,
            pltpu.SemaphoreType.DMA,
            pltpu.SemaphoreType.DMA,
            pltpu.SemaphoreType.DMA,
        ],
        compiler_params=pltpu.CompilerParams(needs_layout_passes=False),
    )
    def k(xi_hbm, t3_hbm, out_hbm,
          xsp, plane, ibuf0, ibuf1, obuf0, obuf1,
          isem0, isem1, wsem0, wsem1):
        cid = lax.axis_index("c")
        sid = lax.axis_index("s")
        wid = sid * _NC + cid

        ibuf = (ibuf0, ibuf1)
        obuf = (obuf0, obuf1)
        isem = (isem0, isem1)
        wsem = (wsem0, wsem1)

        def stage_quarter(q):
            plsc.subcore_barrier()

            @pl.when(sid < 2)
            def _big():
                base = sid * 4
                pltpu.sync_copy(xi_hbm.at[pl.ds(_MQ * q + base, 4)],
                                xsp.at[pl.ds(base, 4)])

            @pl.when(sid >= 2)
            def _small():
                base = sid * 3 + 2
                pltpu.sync_copy(xi_hbm.at[pl.ds(_MQ * q + base, 3)],
                                xsp.at[pl.ds(base, 3)])

            plsc.subcore_barrier()

        def issue_idx(m, p):
            pltpu.async_copy(xsp.at[m], ibuf[p], isem[p])

        def wait_idx(p):
            pltpu.make_async_copy(xsp.at[0], ibuf[p], isem[p]).wait()

        def issue_wb(d, mg, p):
            pltpu.async_copy(obuf[p], out_hbm.at[d, mg], wsem[p])

        def wait_wb(p):
            pltpu.make_async_copy(obuf[p], out_hbm.at[0, 0], wsem[p]).wait()

        def gather_sub(p):
            def body(i, carry):
                for j in range(8):
                    idxv = ibuf[p][i, pl.ds(16 * j, 16)]
                    q = lax.shift_right_logical(idxv, 7)
                    l = lax.bitwise_and(idxv, 127)
                    obuf[p][i, pl.ds(16 * j, 16)] = plsc.load_gather(
                        plane, [q, l])
                return carry

            lax.fori_loop(0, _SUB, body, 0)

        def run_quarter(d, qq, active):
            def work():
                issue_idx(0, 0)
                issue_idx(1, 1)

                def sub_body(m, carry2):
                    def go(p):
                        wait_idx(p)

                        @pl.when(m >= 2)
                        def _():
                            wait_wb(p)

                        gather_sub(p)
                        issue_wb(d, _MQ * qq + m, p)

                        @pl.when(m + 2 < _MQ)
                        def _():
                            issue_idx(m + 2, p)

                    @pl.when(lax.rem(m, 2) == 0)
                    def _():
                        go(0)

                    @pl.when(lax.rem(m, 2) == 1)
                    def _():
                        go(1)

                    return carry2

                lax.fori_loop(0, _MQ, sub_body, 0)
                wait_wb(0)
                wait_wb(1)

            pl.when(active)(work)

        def do_plane(pi, carry):
            d = wid + pi * _NW
            active = d < DIM

            @pl.when(active)
            def _load():
                pltpu.sync_copy(t3_hbm.at[d], plane)

            for qq in range(_NQ):
                stage_quarter(qq)
                run_quarter(d, qq, active)

            return carry

        lax.fori_loop(0, _PLANES_MAX, do_plane, 0)

    return k(xi, t3)


def kernel(x, table):
    xt = x.T.astype(jnp.int32)
    xi = (xt.reshape(25, 8, 32, 128)
          .transpose(0, 2, 1, 3)
          .reshape(_M, _SUB, 128))
    t3 = jnp.pad(table.T, ((0, 4), (0, _QROWS * 128 - NUM_EMB))
                 ).reshape(DIM + 4, _QROWS, 128)
    o4 = _sc_plane_gather(xi, t3)
    out = (o4.reshape(DIM, 25, 8, 4, 8, 128)
           .reshape(DIM, 25, 32, 8, 128)
           .transpose(2, 4, 1, 3, 0)
           .reshape(B, T, DIM))
    return out


# speedup vs baseline: 2.3779x; 2.3779x over previous
import functools

import jax
import jax.numpy as jnp
from jax import lax
from jax.experimental import pallas as pl
from jax.experimental.pallas import tpu as pltpu
from jax.experimental.pallas import tpu_sc as plsc

NUM_EMB = 100000
DIM = 300
B = 4096
T = 200

_info = plsc.get_sparse_core_info()
_NC, _NS = _info.num_cores, _info.num_subcores
_NW = _NC * _NS

_QROWS = 782
_M = 200
_SUB = 32
_NQ = 4
_MQ = _M // _NQ
_PLANES_MAX = -(-DIM // _NW)


def _sc_plane_gather(xi, t3):
    mesh = plsc.VectorSubcoreMesh(core_axis_name="c", subcore_axis_name="s")

    @functools.partial(
        pl.kernel,
        out_type=jax.ShapeDtypeStruct((DIM, _M, _SUB, 128), jnp.float32),
        mesh=mesh,
        scratch_types=[
            pltpu.VMEM_SHARED((_MQ, _SUB, 128), jnp.int32),
            pltpu.VMEM((_QROWS, 128), jnp.float32),
            pltpu.VMEM((_SUB, 128), jnp.int32),
            pltpu.VMEM((_SUB, 128), jnp.int32),
            pltpu.VMEM((_SUB, 128), jnp.float32),
            pltpu.VMEM((_SUB, 128), jnp.float32),
            pltpu.SemaphoreType.DMA,
            pltpu.SemaphoreType.DMA,
            pltpu.SemaphoreType.DMA,
            pltpu.SemaphoreType.DMA,
        ],
        compiler_params=pltpu.CompilerParams(needs_layout_passes=False),
    )
    def k(xi_hbm, t3_hbm, out_hbm,
          xsp, plane, ibuf0, ibuf1, obuf0, obuf1,
          isem0, isem1, wsem0, wsem1):
        cid = lax.axis_index("c")
        sid = lax.axis_index("s")
        wid = sid * _NC + cid

        ibuf = (ibuf0, ibuf1)
        obuf = (obuf0, obuf1)
        isem = (isem0, isem1)
        wsem = (wsem0, wsem1)

        def stage_quarter(q):
            plsc.subcore_barrier()

            @pl.when(sid < 2)
            def _big():
                base = sid * 4
                pltpu.sync_copy(xi_hbm.at[pl.ds(_MQ * q + base, 4)],
                                xsp.at[pl.ds(base, 4)])

            @pl.when(sid >= 2)
            def _small():
                base = sid * 3 + 2
                pltpu.sync_copy(xi_hbm.at[pl.ds(_MQ * q + base, 3)],
                                xsp.at[pl.ds(base, 3)])

            plsc.subcore_barrier()

        def issue_idx(m, p):
            pltpu.async_copy(xsp.at[m], ibuf[p], isem[p])

        def wait_idx(p):
            pltpu.make_async_copy(xsp.at[0], ibuf[p], isem[p]).wait()

        def issue_wb(d, mg, p):
            pltpu.async_copy(obuf[p], out_hbm.at[d, mg], wsem[p])

        def wait_wb(p):
            pltpu.make_async_copy(obuf[p], out_hbm.at[0, 0], wsem[p]).wait()

        def gather_sub(p):
            @plsc.parallel_loop(0, _SUB, 1, unroll=2)
            def _body(i):
                idxs = [ibuf[p][i, pl.ds(16 * j, 16)] for j in range(8)]
                vals = [plsc.load_gather(
                    plane,
                    [lax.shift_right_logical(v, 7), lax.bitwise_and(v, 127)])
                    for v in idxs]
                for j in range(8):
                    obuf[p][i, pl.ds(16 * j, 16)] = vals[j]

        def run_quarter(d, qq, active):
            def work():
                issue_idx(0, 0)
                issue_idx(1, 1)

                def sub_body(m, carry2):
                    def go(p):
                        wait_idx(p)

                        @pl.when(m >= 2)
                        def _():
                            wait_wb(p)

                        gather_sub(p)
                        issue_wb(d, _MQ * qq + m, p)

                        @pl.when(m + 2 < _MQ)
                        def _():
                            issue_idx(m + 2, p)

                    @pl.when(lax.rem(m, 2) == 0)
                    def _():
                        go(0)

                    @pl.when(lax.rem(m, 2) == 1)
                    def _():
                        go(1)

                    return carry2

                lax.fori_loop(0, _MQ, sub_body, 0)
                wait_wb(0)
                wait_wb(1)

            pl.when(active)(work)

        def do_plane(pi, carry):
            d = wid + pi * _NW
            active = d < DIM

            @pl.when(active)
            def _load():
                pltpu.sync_copy(t3_hbm.at[d], plane)

            for qq in range(_NQ):
                stage_quarter(qq)
                run_quarter(d, qq, active)

            return carry

        lax.fori_loop(0, _PLANES_MAX, do_plane, 0)

    return k(xi, t3)


def kernel(x, table):
    xt = x.T.astype(jnp.int32)
    xi = (xt.reshape(25, 8, 32, 128)
          .transpose(0, 2, 1, 3)
          .reshape(_M, _SUB, 128))
    t3 = jnp.pad(table.T, ((0, 4), (0, _QROWS * 128 - NUM_EMB))
                 ).reshape(DIM + 4, _QROWS, 128)
    o4 = _sc_plane_gather(xi, t3)
    out = (o4.reshape(DIM, 25, 8, 4, 8, 128)
           .reshape(DIM, 25, 32, 8, 128)
           .transpose(2, 4, 1, 3, 0)
           .reshape(B, T, DIM))
    return out


# speedup vs baseline: 2.4347x; 1.0239x over previous
import functools

import jax
import jax.numpy as jnp
from jax import lax
from jax.experimental import pallas as pl
from jax.experimental.pallas import tpu as pltpu
from jax.experimental.pallas import tpu_sc as plsc

NUM_EMB = 100000
DIM = 300
B = 4096
T = 200

_info = plsc.get_sparse_core_info()
_NC, _NS = _info.num_cores, _info.num_subcores
_NW = _NC * _NS

_QROWS = 782
_M = 200
_SUB = 32
_NQ = 4
_MQ = _M // _NQ
_PLANES_MAX = -(-DIM // _NW)


def _sc_plane_gather(xi, t3):
    mesh = plsc.VectorSubcoreMesh(core_axis_name="c", subcore_axis_name="s")

    @functools.partial(
        pl.kernel,
        out_type=jax.ShapeDtypeStruct((DIM, _M, _SUB, 128), jnp.float32),
        mesh=mesh,
        scratch_types=[
            pltpu.VMEM_SHARED((_MQ, _SUB, 128), jnp.int32),
            pltpu.VMEM((_QROWS, 128), jnp.float32),
            pltpu.VMEM((_SUB, 128), jnp.int32),
            pltpu.VMEM((_SUB, 128), jnp.int32),
            pltpu.VMEM((_SUB, 128), jnp.float32),
            pltpu.VMEM((_SUB, 128), jnp.float32),
            pltpu.SemaphoreType.DMA,
            pltpu.SemaphoreType.DMA,
            pltpu.SemaphoreType.DMA,
            pltpu.SemaphoreType.DMA,
        ],
        compiler_params=pltpu.CompilerParams(needs_layout_passes=False),
    )
    def k(xi_hbm, t3_hbm, out_hbm,
          xsp, plane, ibuf0, ibuf1, obuf0, obuf1,
          isem0, isem1, wsem0, wsem1):
        cid = lax.axis_index("c")
        sid = lax.axis_index("s")
        wid = sid * _NC + cid

        ibuf = (ibuf0, ibuf1)
        obuf = (obuf0, obuf1)
        isem = (isem0, isem1)
        wsem = (wsem0, wsem1)

        def stage_quarter(q):
            plsc.subcore_barrier()

            @pl.when(sid < 2)
            def _big():
                base = sid * 4
                pltpu.sync_copy(xi_hbm.at[pl.ds(_MQ * q + base, 4)],
                                xsp.at[pl.ds(base, 4)])

            @pl.when(sid >= 2)
            def _small():
                base = sid * 3 + 2
                pltpu.sync_copy(xi_hbm.at[pl.ds(_MQ * q + base, 3)],
                                xsp.at[pl.ds(base, 3)])

            plsc.subcore_barrier()

        def issue_idx(m, p):
            pltpu.async_copy(xsp.at[m], ibuf[p], isem[p])

        def wait_idx(p):
            pltpu.make_async_copy(xsp.at[0], ibuf[p], isem[p]).wait()

        def issue_wb(d, mg, p):
            pltpu.async_copy(obuf[p], out_hbm.at[d, mg], wsem[p])

        def wait_wb(p):
            pltpu.make_async_copy(obuf[p], out_hbm.at[0, 0], wsem[p]).wait()

        def gather_sub(p):
            @plsc.parallel_loop(0, _SUB, 1, unroll=4)
            def _body(i):
                idxs = [ibuf[p][i, pl.ds(16 * j, 16)] for j in range(8)]
                vals = [plsc.load_gather(
                    plane,
                    [lax.shift_right_logical(v, 7), lax.bitwise_and(v, 127)])
                    for v in idxs]
                for j in range(8):
                    obuf[p][i, pl.ds(16 * j, 16)] = vals[j]

        def run_quarter(d, qq, active):
            def work():
                issue_idx(0, 0)
                issue_idx(1, 1)

                def sub_body(m, carry2):
                    def go(p):
                        wait_idx(p)

                        @pl.when(m >= 2)
                        def _():
                            wait_wb(p)

                        gather_sub(p)
                        issue_wb(d, _MQ * qq + m, p)

                        @pl.when(m + 2 < _MQ)
                        def _():
                            issue_idx(m + 2, p)

                    @pl.when(lax.rem(m, 2) == 0)
                    def _():
                        go(0)

                    @pl.when(lax.rem(m, 2) == 1)
                    def _():
                        go(1)

                    return carry2

                lax.fori_loop(0, _MQ, sub_body, 0)
                wait_wb(0)
                wait_wb(1)

            pl.when(active)(work)

        def do_plane(pi, carry):
            d = wid + pi * _NW
            active = d < DIM

            @pl.when(active)
            def _load():
                pltpu.sync_copy(t3_hbm.at[d], plane)

            for qq in range(_NQ):
                stage_quarter(qq)
                run_quarter(d, qq, active)

            return carry

        lax.fori_loop(0, _PLANES_MAX, do_plane, 0)

    return k(xi, t3)


def kernel(x, table):
    xt = x.T.astype(jnp.int32)
    xi = (xt.reshape(25, 8, 32, 128)
          .transpose(0, 2, 1, 3)
          .reshape(_M, _SUB, 128))
    t3 = jnp.pad(table.T, ((0, 4), (0, _QROWS * 128 - NUM_EMB))
                 ).reshape(DIM + 4, _QROWS, 128)
    o4 = _sc_plane_gather(xi, t3)
    out = (o4.reshape(DIM, 25, 8, 4, 8, 128)
           .reshape(DIM, 25, 32, 8, 128)
           .transpose(2, 4, 1, 3, 0)
           .reshape(B, T, DIM))
    return out
